# baseline (speedup 1.0000x reference)
import jax
import jax.numpy as jnp
from jax import lax
from jax.experimental import pallas as pl
from jax.experimental.pallas import tpu as pltpu
from jax.experimental.pallas import tpu_sc as plsc

NC = 2
NS = 16
NW = NC * NS

B = 16384
L = 50
D = 64
TB = B // 128
TB_W = TB // NW
NCHUNK = L * TB_W
NBUF = 2


def _transpose_chunk(gbuf, tbuf, iota16):
    @pl.loop(0, D)
    def _(d):
        cols = jnp.full((16,), d, jnp.int32)
        td = d // 8
        r = d - td * 8
        for cb in range(8):
            v = plsc.load_gather(gbuf, [iota16 + (cb * 16), cols])
            tbuf[td, r, pl.ds(cb * 16, 16)] = v


def _body(table_hbm, xt_hbm, p_hbm, idx_v, gbuf, tbuf, gsems, osems):
    wid = lax.axis_index("s") * NC + lax.axis_index("c")
    iota16 = lax.broadcasted_iota(jnp.int32, (16,), 0)
    pltpu.sync_copy(xt_hbm.at[:, pl.ds(wid * 512, 512)], idx_v)

    def chunk_lt(k):
        l = k // TB_W
        t = k - l * TB_W
        return l, t

    def gather(k, b):
        l, t = chunk_lt(k)
        pltpu.async_copy(
            table_hbm.at[idx_v.at[l, pl.ds(t * 128, 128)]],
            gbuf.at[b],
            gsems.at[b],
        )

    def writeback(k, b):
        l, t = chunk_lt(k)
        return pltpu.make_async_copy(
            tbuf.at[b], p_hbm.at[l, :, wid * TB_W + t], osems.at[b]
        )

    for b in range(NBUF):
        gather(b, b)

    @pl.loop(0, NCHUNK, step=NBUF)
    def _(j):
        for b in range(NBUF):
            k = j + b
            l, t = chunk_lt(k)
            pltpu.make_async_copy(
                table_hbm.at[idx_v.at[l, pl.ds(t * 128, 128)]],
                gbuf.at[b],
                gsems.at[b],
            ).wait()

            @pl.when(k >= NBUF)
            def _():
                lp, tp = chunk_lt(k - NBUF)
                pltpu.make_async_copy(
                    tbuf.at[b], p_hbm.at[lp, :, wid * TB_W + tp], osems.at[b]
                ).wait()

            _transpose_chunk(gbuf.at[b], tbuf.at[b], iota16)
            writeback(k, b).start()

            @pl.when(k + NBUF < NCHUNK)
            def _():
                gather(k + NBUF, b)

    for b in range(NBUF):
        writeback(NCHUNK - NBUF + b, b).wait()


@jax.jit
def _gather(table, xt):
    mesh = plsc.VectorSubcoreMesh(
        core_axis_name="c", subcore_axis_name="s", num_cores=NC, num_subcores=NS
    )
    return pl.kernel(
        _body,
        out_type=jax.ShapeDtypeStruct((L, 8, TB, 8, 128), jnp.float32),
        mesh=mesh,
        scratch_types=[
            pltpu.VMEM((L, 512), jnp.int32),
            pltpu.VMEM((NBUF, 128, D), jnp.float32),
            pltpu.VMEM((NBUF, 8, 8, 128), jnp.float32),
            pltpu.SemaphoreType.DMA((NBUF,)),
            pltpu.SemaphoreType.DMA((NBUF,)),
        ],
        compiler_params=pltpu.CompilerParams(
            use_tc_tiling_on_sc=False, needs_layout_passes=False
        ),
    )(table, xt)


def kernel(x, table):
    xt = jnp.transpose(x.astype(jnp.int32))
    p = _gather(table, xt)
    return p.transpose(2, 4, 0, 1, 3).reshape(B, L, D)


# speedup vs baseline: 1.6139x; 1.6139x over previous
import jax
import jax.numpy as jnp
from jax import lax
from jax.experimental import pallas as pl
from jax.experimental.pallas import tpu as pltpu
from jax.experimental.pallas import tpu_sc as plsc

NC = 2
NS = 16
NW = NC * NS

B = 16384
L = 50
D = 64
TB = B // 128
TB_W = TB // NW
NCHUNK = L * TB_W
NBUF = 2


TPITCH = 137


def _transpose_chunk(gbuf, tbuf, drows):
    @pl.loop(0, 128, unroll=2)
    def _(c):
        cols = jnp.full((16,), c, jnp.int32)
        for d0 in range(4):
            v = gbuf[c, pl.ds(d0 * 16, 16)]
            plsc.store_scatter(tbuf, [drows[d0], cols], v)


def _body(table_hbm, xt_hbm, p_hbm, idx_v, gbuf, tbuf, gsems, osems):
    wid = lax.axis_index("s") * NC + lax.axis_index("c")
    iota16 = lax.broadcasted_iota(jnp.int32, (16,), 0)
    drows = [iota16 + d0 * 16 for d0 in range(4)]
    pltpu.sync_copy(xt_hbm.at[:, pl.ds(wid * 512, 512)], idx_v)

    def chunk_lt(k):
        l = k // TB_W
        t = k - l * TB_W
        return l, t

    def gather(k, b):
        l, t = chunk_lt(k)
        pltpu.async_copy(
            table_hbm.at[idx_v.at[l, pl.ds(t * 128, 128)]],
            gbuf.at[b],
            gsems.at[b],
        )

    def writeback_all(k, b):
        l, t = chunk_lt(k)
        for td in range(8):
            pltpu.async_copy(
                tbuf.at[b, pl.ds(td * 8, 8), pl.ds(0, 128)],
                p_hbm.at[l, td, wid * TB_W + t],
                osems.at[b],
            )

    def drain_all(k, b):
        l, t = chunk_lt(k)
        for td in range(8):
            pltpu.make_async_copy(
                tbuf.at[b, pl.ds(td * 8, 8), pl.ds(0, 128)],
                p_hbm.at[l, td, wid * TB_W + t],
                osems.at[b],
            ).wait()

    for b in range(NBUF):
        gather(b, b)

    @pl.loop(0, NCHUNK, step=NBUF)
    def _(j):
        for b in range(NBUF):
            k = j + b
            l, t = chunk_lt(k)
            pltpu.make_async_copy(
                table_hbm.at[idx_v.at[l, pl.ds(t * 128, 128)]],
                gbuf.at[b],
                gsems.at[b],
            ).wait()

            @pl.when(k >= NBUF)
            def _():
                drain_all(k - NBUF, b)

            _transpose_chunk(gbuf.at[b], tbuf.at[b], drows)
            writeback_all(k, b)

            @pl.when(k + NBUF < NCHUNK)
            def _():
                gather(k + NBUF, b)

    for b in range(NBUF):
        drain_all(NCHUNK - NBUF + b, b)


@jax.jit
def _gather(table, xt):
    mesh = plsc.VectorSubcoreMesh(
        core_axis_name="c", subcore_axis_name="s", num_cores=NC, num_subcores=NS
    )
    return pl.kernel(
        _body,
        out_type=jax.ShapeDtypeStruct((L, 8, TB, 8, 128), jnp.float32),
        mesh=mesh,
        scratch_types=[
            pltpu.VMEM((L, 512), jnp.int32),
            pltpu.VMEM((NBUF, 128, D), jnp.float32),
            pltpu.VMEM((NBUF, D, TPITCH), jnp.float32),
            pltpu.SemaphoreType.DMA((NBUF,)),
            pltpu.SemaphoreType.DMA((NBUF,)),
        ],
        compiler_params=pltpu.CompilerParams(
            use_tc_tiling_on_sc=False, needs_layout_passes=False
        ),
    )(table, xt)


def kernel(x, table):
    xt = jnp.transpose(x.astype(jnp.int32))
    p = _gather(table, xt)
    return p.transpose(2, 4, 0, 1, 3).reshape(B, L, D)
